# baseline (speedup 1.0000x reference)
import jax
import jax.numpy as jnp
from jax import lax
from jax.experimental import pallas as pl
from jax.experimental.pallas import tpu as pltpu
from jax.experimental.pallas import tpu_sc as plsc

VOCAB = 100000
HIDDEN = 768
SEG_NUM = 3
B, L = 1024, 200
N = B * L

_INFO = plsc.get_sparse_core_info()
NC, NS, LANES = _INFO.num_cores, _INFO.num_subcores, _INFO.num_lanes
NW = NC * NS
PER_W = N // NW
CHUNK = 16
NCHUNKS = PER_W // CHUNK
UNROLL = 8


def _body(token_hbm, segment_hbm, token_tab, pos_tab, seg_tab, out_hbm,
          tok_idx, seg_idx, seg_v,
          buf_a0, buf_a1, buf_b0, buf_b1, out0, out1,
          sem_g0, sem_g1, sem_w0, sem_w1):
    wid = lax.axis_index("s") * NC + lax.axis_index("c")
    base = wid * PER_W
    bufs_a = (buf_a0, buf_a1)
    bufs_b = (buf_b0, buf_b1)
    outs = (out0, out1)
    sems_g = (sem_g0, sem_g1)
    sems_w = (sem_w0, sem_w1)

    pltpu.sync_copy(token_hbm.at[pl.ds(base, PER_W)], tok_idx)
    pltpu.sync_copy(segment_hbm.at[pl.ds(base, PER_W)], seg_idx)
    pltpu.sync_copy(seg_tab, seg_v)

    def fire_gathers(chunk, p):
        idx = tok_idx.at[pl.ds(chunk * CHUNK, CHUNK)]
        pltpu.async_copy(token_tab.at[idx], bufs_a[p], sems_g[p])
        pltpu.async_copy(pos_tab.at[idx], bufs_b[p], sems_g[p])

    fire_gathers(0, 0)
    fire_gathers(1, 1)

    rowvec = lax.iota(jnp.int32, LANES)

    def process(chunk, p):
        pltpu.make_async_copy(token_tab.at[pl.ds(0, CHUNK)], bufs_a[p],
                              sems_g[p]).wait()
        pltpu.make_async_copy(token_tab.at[pl.ds(0, CHUNK)], bufs_b[p],
                              sems_g[p]).wait()
        @pl.when(chunk >= 2)
        def _():
            pltpu.make_async_copy(out_hbm.at[pl.ds(base, CHUNK)], outs[p],
                                  sems_w[p]).wait()

        svec = seg_idx[pl.ds(chunk * CHUNK, LANES)]

        def col_step(i, _):
            c0 = i * UNROLL
            for k in range(UNROLL):
                cv = jnp.full((LANES,), c0 + k, jnp.int32)
                va = plsc.load_gather(bufs_a[p], [rowvec, cv])
                vb = plsc.load_gather(bufs_b[p], [rowvec, cv])
                vs = plsc.load_gather(seg_v, [svec, cv])
                plsc.store_scatter(outs[p], [rowvec, cv], va + vb + vs)
            return 0

        lax.fori_loop(0, HIDDEN // UNROLL, col_step, 0)

        pltpu.async_copy(outs[p], out_hbm.at[pl.ds(base + chunk * CHUNK, CHUNK)],
                         sems_w[p])

        @pl.when(chunk + 2 < NCHUNKS)
        def _():
            fire_gathers(chunk + 2, p)

    def pair_step(i, _):
        process(2 * i, 0)
        process(2 * i + 1, 1)
        return 0

    lax.fori_loop(0, NCHUNKS // 2, pair_step, 0)

    pltpu.make_async_copy(out_hbm.at[pl.ds(base, CHUNK)], outs[0],
                          sems_w[0]).wait()
    pltpu.make_async_copy(out_hbm.at[pl.ds(base, CHUNK)], outs[1],
                          sems_w[1]).wait()


@jax.jit
def _run(token_flat, segment_flat, token_table, pos_table, seg_table):
    mesh = plsc.VectorSubcoreMesh(core_axis_name="c", subcore_axis_name="s")
    kern = pl.kernel(
        _body,
        out_type=jax.ShapeDtypeStruct((N, HIDDEN), jnp.float32),
        mesh=mesh,
        compiler_params=pltpu.CompilerParams(use_tc_tiling_on_sc=False,
                                             needs_layout_passes=False),
        scratch_types=[
            pltpu.VMEM((PER_W,), jnp.int32),
            pltpu.VMEM((PER_W,), jnp.int32),
            pltpu.VMEM((SEG_NUM, HIDDEN), jnp.float32),
            pltpu.VMEM((CHUNK, HIDDEN), jnp.float32),
            pltpu.VMEM((CHUNK, HIDDEN), jnp.float32),
            pltpu.VMEM((CHUNK, HIDDEN), jnp.float32),
            pltpu.VMEM((CHUNK, HIDDEN), jnp.float32),
            pltpu.VMEM((CHUNK, HIDDEN), jnp.float32),
            pltpu.VMEM((CHUNK, HIDDEN), jnp.float32),
            pltpu.SemaphoreType.DMA,
            pltpu.SemaphoreType.DMA,
            pltpu.SemaphoreType.DMA,
            pltpu.SemaphoreType.DMA,
        ],
    )
    return kern(token_flat, segment_flat, token_table, pos_table, seg_table)


def kernel(token, segment, token_table, pos_table, seg_table):
    token_flat = token.reshape(N).astype(jnp.int32)
    segment_flat = segment.reshape(N).astype(jnp.int32)
    out = _run(token_flat, segment_flat, token_table, pos_table, seg_table)
    return out.reshape(B, L, HIDDEN)


# speedup vs baseline: 6.7651x; 6.7651x over previous
import jax
import jax.numpy as jnp
from jax import lax
from jax.experimental import pallas as pl
from jax.experimental.pallas import tpu as pltpu
from jax.experimental.pallas import tpu_sc as plsc

VOCAB = 100000
HIDDEN = 768
SEG_NUM = 3
B, L = 1024, 200
N = B * L

_INFO = plsc.get_sparse_core_info()
NC, NS, LANES = _INFO.num_cores, _INFO.num_subcores, _INFO.num_lanes
NW = NC * NS
PER_W = N // NW
CHUNK = 16
NCHUNKS = PER_W // CHUNK
CBLKS = HIDDEN // LANES


def _body(token_hbm, segment_hbm, token_tab, pos_tab, seg_tab, out_hbm,
          tok_idx, seg_idx, seg_v, d1_v, d2_v,
          buf_a0, buf_a1, buf_b0, buf_b1, out0, out1,
          sem_g0, sem_g1, sem_w0, sem_w1):
    wid = lax.axis_index("s") * NC + lax.axis_index("c")
    base = wid * PER_W
    bufs_a = (buf_a0, buf_a1)
    bufs_b = (buf_b0, buf_b1)
    outs = (out0, out1)
    sems_g = (sem_g0, sem_g1)
    sems_w = (sem_w0, sem_w1)

    pltpu.sync_copy(token_hbm.at[pl.ds(base, PER_W)], tok_idx)
    pltpu.sync_copy(segment_hbm.at[pl.ds(base, PER_W)], seg_idx)
    pltpu.sync_copy(seg_tab, seg_v)

    def delta_step(j, _):
        sl = pl.ds(j * LANES, LANES)
        r0 = seg_v[0, sl]
        r1 = seg_v[1, sl]
        r2 = seg_v[2, sl]
        d1_v[sl] = r1 - r0
        d2_v[sl] = r2 - r1
        return 0

    lax.fori_loop(0, CBLKS, delta_step, 0)

    def fire_gathers(chunk, p):
        idx = tok_idx.at[pl.ds(chunk * CHUNK, CHUNK)]
        pltpu.async_copy(token_tab.at[idx], bufs_a[p], sems_g[p])
        pltpu.async_copy(pos_tab.at[idx], bufs_b[p], sems_g[p])

    fire_gathers(0, 0)
    fire_gathers(1, 1)

    def process(chunk, p):
        pltpu.make_async_copy(token_tab.at[pl.ds(0, CHUNK)], bufs_a[p],
                              sems_g[p]).wait()
        pltpu.make_async_copy(token_tab.at[pl.ds(0, CHUNK)], bufs_b[p],
                              sems_g[p]).wait()
        @pl.when(chunk >= 2)
        def _():
            pltpu.make_async_copy(out_hbm.at[pl.ds(base, CHUNK)], outs[p],
                                  sems_w[p]).wait()

        svec = seg_idx[pl.ds(chunk * CHUNK, LANES)]
        m1 = (svec >= 1).astype(jnp.float32)
        m2 = (svec >= 2).astype(jnp.float32)

        def col_step(j, _):
            sl = pl.ds(j * LANES, LANES)
            s0 = seg_v[0, sl]
            d1 = d1_v[sl]
            d2 = d2_v[sl]
            for r in range(CHUNK):
                m1b = jnp.broadcast_to(m1[r], (LANES,))
                m2b = jnp.broadcast_to(m2[r], (LANES,))
                seg_row = s0 + m1b * d1 + m2b * d2
                outs[p][r, sl] = bufs_a[p][r, sl] + bufs_b[p][r, sl] + seg_row
            return 0

        lax.fori_loop(0, CBLKS, col_step, 0)

        pltpu.async_copy(outs[p], out_hbm.at[pl.ds(base + chunk * CHUNK, CHUNK)],
                         sems_w[p])

        @pl.when(chunk + 2 < NCHUNKS)
        def _():
            fire_gathers(chunk + 2, p)

    def pair_step(i, _):
        process(2 * i, 0)
        process(2 * i + 1, 1)
        return 0

    lax.fori_loop(0, NCHUNKS // 2, pair_step, 0)

    pltpu.make_async_copy(out_hbm.at[pl.ds(base, CHUNK)], outs[0],
                          sems_w[0]).wait()
    pltpu.make_async_copy(out_hbm.at[pl.ds(base, CHUNK)], outs[1],
                          sems_w[1]).wait()


@jax.jit
def _run(token_flat, segment_flat, token_table, pos_table, seg_table):
    mesh = plsc.VectorSubcoreMesh(core_axis_name="c", subcore_axis_name="s")
    kern = pl.kernel(
        _body,
        out_type=jax.ShapeDtypeStruct((N, HIDDEN), jnp.float32),
        mesh=mesh,
        compiler_params=pltpu.CompilerParams(use_tc_tiling_on_sc=False,
                                             needs_layout_passes=False),
        scratch_types=[
            pltpu.VMEM((PER_W,), jnp.int32),
            pltpu.VMEM((PER_W,), jnp.int32),
            pltpu.VMEM((SEG_NUM, HIDDEN), jnp.float32),
            pltpu.VMEM((HIDDEN,), jnp.float32),
            pltpu.VMEM((HIDDEN,), jnp.float32),
            pltpu.VMEM((CHUNK, HIDDEN), jnp.float32),
            pltpu.VMEM((CHUNK, HIDDEN), jnp.float32),
            pltpu.VMEM((CHUNK, HIDDEN), jnp.float32),
            pltpu.VMEM((CHUNK, HIDDEN), jnp.float32),
            pltpu.VMEM((CHUNK, HIDDEN), jnp.float32),
            pltpu.VMEM((CHUNK, HIDDEN), jnp.float32),
            pltpu.SemaphoreType.DMA,
            pltpu.SemaphoreType.DMA,
            pltpu.SemaphoreType.DMA,
            pltpu.SemaphoreType.DMA,
        ],
    )
    return kern(token_flat, segment_flat, token_table, pos_table, seg_table)


def kernel(token, segment, token_table, pos_table, seg_table):
    token_flat = token.reshape(N).astype(jnp.int32)
    segment_flat = segment.reshape(N).astype(jnp.int32)
    out = _run(token_flat, segment_flat, token_table, pos_table, seg_table)
    return out.reshape(B, L, HIDDEN)


# speedup vs baseline: 16.2848x; 2.4072x over previous
import jax
import jax.numpy as jnp
from jax import lax
from jax.experimental import pallas as pl
from jax.experimental.pallas import tpu as pltpu
from jax.experimental.pallas import tpu_sc as plsc

VOCAB = 100000
HIDDEN = 768
SEG_NUM = 3
B, L = 1024, 200
N = B * L

_INFO = plsc.get_sparse_core_info()
NC, NS, LANES = _INFO.num_cores, _INFO.num_subcores, _INFO.num_lanes
NW = NC * NS
PER_W = N // NW
CHUNK = 16
NCHUNKS = PER_W // CHUNK
CBLKS = HIDDEN // LANES


def _body(token_hbm, segment_hbm, token_tab, pos_tab, seg_tab, out_hbm,
          tok_idx, seg_idx, seg_v, d1_v, d2_v,
          buf_a0, buf_a1, buf_b0, buf_b1, out0, out1,
          sem_g0, sem_g1, sem_w0, sem_w1):
    wid = lax.axis_index("s") * NC + lax.axis_index("c")
    base = wid * PER_W
    bufs_a = (buf_a0, buf_a1)
    bufs_b = (buf_b0, buf_b1)
    outs = (out0, out1)
    sems_g = (sem_g0, sem_g1)
    sems_w = (sem_w0, sem_w1)

    pltpu.sync_copy(token_hbm.at[pl.ds(base, PER_W)], tok_idx)
    pltpu.sync_copy(segment_hbm.at[pl.ds(base, PER_W)], seg_idx)
    pltpu.sync_copy(seg_tab, seg_v)

    def delta_step(j, _):
        sl = pl.ds(j * LANES, LANES)
        r0 = seg_v[0, sl]
        r1 = seg_v[1, sl]
        r2 = seg_v[2, sl]
        d1_v[sl] = r1 - r0
        d2_v[sl] = r2 - r1
        return 0

    lax.fori_loop(0, CBLKS, delta_step, 0)

    def fire_gathers(chunk, p):
        idx = tok_idx.at[pl.ds(chunk * CHUNK, CHUNK)]
        pltpu.async_copy(token_tab.at[idx], bufs_a[p], sems_g[p])
        pltpu.async_copy(pos_tab.at[idx], bufs_b[p], sems_g[p])

    fire_gathers(0, 0)
    fire_gathers(1, 1)

    def process(chunk, p):
        pltpu.make_async_copy(token_tab.at[pl.ds(0, CHUNK)], bufs_a[p],
                              sems_g[p]).wait()
        pltpu.make_async_copy(token_tab.at[pl.ds(0, CHUNK)], bufs_b[p],
                              sems_g[p]).wait()
        @pl.when(chunk >= 2)
        def _():
            pltpu.make_async_copy(out_hbm.at[pl.ds(base, CHUNK)], outs[p],
                                  sems_w[p]).wait()

        svec = seg_idx[pl.ds(chunk * CHUNK, LANES)]
        m1 = (svec >= 1).astype(jnp.float32)
        m2 = (svec >= 2).astype(jnp.float32)

        def col_step(j, _):
            sl = pl.ds(j * LANES, LANES)
            s0 = seg_v[0, sl]
            d1 = d1_v[sl]
            d2 = d2_v[sl]
            for r in range(CHUNK):
                m1b = jnp.broadcast_to(m1[r], (LANES,))
                m2b = jnp.broadcast_to(m2[r], (LANES,))
                seg_row = s0 + m1b * d1 + m2b * d2
                outs[p][r, sl] = bufs_a[p][r, sl] + bufs_b[p][r, sl] + seg_row
            return 0

        lax.fori_loop(0, CBLKS, col_step, 0)

        pltpu.async_copy(outs[p], out_hbm.at[pl.ds(base + chunk * CHUNK, CHUNK)],
                         sems_w[p])

        @pl.when(chunk + 2 < NCHUNKS)
        def _():
            fire_gathers(chunk + 2, p)

    def pair_step(i, _):
        process(2 * i, 0)
        process(2 * i + 1, 1)
        return 0

    lax.fori_loop(0, NCHUNKS // 2, pair_step, 0)

    pltpu.make_async_copy(out_hbm.at[pl.ds(base, CHUNK)], outs[0],
                          sems_w[0]).wait()
    pltpu.make_async_copy(out_hbm.at[pl.ds(base, CHUNK)], outs[1],
                          sems_w[1]).wait()


@jax.jit
def _run(token_flat, segment_flat, token_table, pos_table, seg_table):
    mesh = plsc.VectorSubcoreMesh(core_axis_name="c", subcore_axis_name="s")
    kern = pl.kernel(
        _body,
        out_type=jax.ShapeDtypeStruct((N, HIDDEN), jnp.float32),
        mesh=mesh,
        compiler_params=pltpu.CompilerParams(needs_layout_passes=False),
        scratch_types=[
            pltpu.VMEM((PER_W,), jnp.int32),
            pltpu.VMEM((PER_W,), jnp.int32),
            pltpu.VMEM((8, HIDDEN), jnp.float32),
            pltpu.VMEM((HIDDEN,), jnp.float32),
            pltpu.VMEM((HIDDEN,), jnp.float32),
            pltpu.VMEM((CHUNK, HIDDEN), jnp.float32),
            pltpu.VMEM((CHUNK, HIDDEN), jnp.float32),
            pltpu.VMEM((CHUNK, HIDDEN), jnp.float32),
            pltpu.VMEM((CHUNK, HIDDEN), jnp.float32),
            pltpu.VMEM((CHUNK, HIDDEN), jnp.float32),
            pltpu.VMEM((CHUNK, HIDDEN), jnp.float32),
            pltpu.SemaphoreType.DMA,
            pltpu.SemaphoreType.DMA,
            pltpu.SemaphoreType.DMA,
            pltpu.SemaphoreType.DMA,
        ],
    )
    return kern(token_flat, segment_flat, token_table, pos_table, seg_table)


def kernel(token, segment, token_table, pos_table, seg_table):
    token_flat = token.reshape(N).astype(jnp.int32)
    segment_flat = segment.reshape(N).astype(jnp.int32)
    seg_pad = jnp.concatenate(
        [seg_table, jnp.zeros((8 - SEG_NUM, HIDDEN), jnp.float32)], axis=0)
    out = _run(token_flat, segment_flat, token_table, pos_table, seg_pad)
    return out.reshape(B, L, HIDDEN)


# speedup vs baseline: 17.7539x; 1.0902x over previous
import jax
import jax.numpy as jnp
from jax import lax
from jax.experimental import pallas as pl
from jax.experimental.pallas import tpu as pltpu
from jax.experimental.pallas import tpu_sc as plsc

VOCAB = 100000
HIDDEN = 768
SEG_NUM = 3
B, L = 1024, 200
N = B * L

_INFO = plsc.get_sparse_core_info()
NC, NS, LANES = _INFO.num_cores, _INFO.num_subcores, _INFO.num_lanes
NW = NC * NS
PER_W = N // NW
CHUNK = 16
NCHUNKS = PER_W // CHUNK
CBLKS = HIDDEN // LANES
NSLOTS = 3


def _body(token_hbm, segment_hbm, token_tab, pos_tab, seg_tab, out_hbm,
          tok_idx, seg_idx, seg_v,
          buf_a0, buf_a1, buf_a2, buf_b0, buf_b1, buf_b2, out0, out1, out2,
          sem_g0, sem_g1, sem_g2, sem_w0, sem_w1, sem_w2):
    wid = lax.axis_index("s") * NC + lax.axis_index("c")
    base = wid * PER_W
    bufs_a = (buf_a0, buf_a1, buf_a2)
    bufs_b = (buf_b0, buf_b1, buf_b2)
    outs = (out0, out1, out2)
    sems_g = (sem_g0, sem_g1, sem_g2)
    sems_w = (sem_w0, sem_w1, sem_w2)

    pltpu.sync_copy(token_hbm.at[pl.ds(base, PER_W)], tok_idx)
    pltpu.sync_copy(segment_hbm.at[pl.ds(base, PER_W)], seg_idx)
    pltpu.sync_copy(seg_tab, seg_v)

    def delta_step(j, _):
        sl = pl.ds(j * LANES, LANES)
        r0 = seg_v[0, sl]
        r1 = seg_v[1, sl]
        r2 = seg_v[2, sl]
        seg_v[2, sl] = r2 - r1
        seg_v[1, sl] = r1 - r0
        return 0

    lax.fori_loop(0, CBLKS, delta_step, 0)

    def fire_gathers(chunk, p):
        idx = tok_idx.at[pl.ds(chunk * CHUNK, CHUNK)]
        pltpu.async_copy(token_tab.at[idx], bufs_a[p], sems_g[p])
        pltpu.async_copy(pos_tab.at[idx], bufs_b[p], sems_g[p])

    for p in range(NSLOTS):
        fire_gathers(p, p)

    def process(chunk, p):
        pltpu.make_async_copy(token_tab.at[pl.ds(0, CHUNK)], bufs_a[p],
                              sems_g[p]).wait()
        pltpu.make_async_copy(token_tab.at[pl.ds(0, CHUNK)], bufs_b[p],
                              sems_g[p]).wait()
        @pl.when(chunk >= NSLOTS)
        def _():
            pltpu.make_async_copy(out_hbm.at[pl.ds(base, CHUNK)], outs[p],
                                  sems_w[p]).wait()

        svec = seg_idx[pl.ds(chunk * CHUNK, LANES)]
        m1 = (svec >= 1).astype(jnp.float32)
        m2 = (svec >= 2).astype(jnp.float32)

        def col_step(j, _):
            sl = pl.ds(j * LANES, LANES)
            s0 = seg_v[0, sl]
            d1 = seg_v[1, sl]
            d2 = seg_v[2, sl]
            for r in range(CHUNK):
                m1b = jnp.broadcast_to(m1[r], (LANES,))
                m2b = jnp.broadcast_to(m2[r], (LANES,))
                seg_row = s0 + m1b * d1 + m2b * d2
                outs[p][r, sl] = bufs_a[p][r, sl] + bufs_b[p][r, sl] + seg_row
            return 0

        lax.fori_loop(0, CBLKS, col_step, 0)

        @pl.when(chunk + NSLOTS < NCHUNKS)
        def _():
            fire_gathers(chunk + NSLOTS, p)

        pltpu.async_copy(outs[p], out_hbm.at[pl.ds(base + chunk * CHUNK, CHUNK)],
                         sems_w[p])

    def trip_step(i, _):
        process(NSLOTS * i, 0)
        process(NSLOTS * i + 1, 1)
        process(NSLOTS * i + 2, 2)
        return 0

    lax.fori_loop(0, NCHUNKS // NSLOTS, trip_step, 0)
    process(NCHUNKS - 1, 0)

    for p in range(NSLOTS):
        pltpu.make_async_copy(out_hbm.at[pl.ds(base, CHUNK)], outs[p],
                              sems_w[p]).wait()


@jax.jit
def _run(token_flat, segment_flat, token_table, pos_table, seg_table):
    mesh = plsc.VectorSubcoreMesh(core_axis_name="c", subcore_axis_name="s")
    kern = pl.kernel(
        _body,
        out_type=jax.ShapeDtypeStruct((N, HIDDEN), jnp.float32),
        mesh=mesh,
        compiler_params=pltpu.CompilerParams(needs_layout_passes=False),
        scratch_types=[
            pltpu.VMEM((PER_W,), jnp.int32),
            pltpu.VMEM((PER_W,), jnp.int32),
            pltpu.VMEM((8, HIDDEN), jnp.float32),
            pltpu.VMEM((CHUNK, HIDDEN), jnp.float32),
            pltpu.VMEM((CHUNK, HIDDEN), jnp.float32),
            pltpu.VMEM((CHUNK, HIDDEN), jnp.float32),
            pltpu.VMEM((CHUNK, HIDDEN), jnp.float32),
            pltpu.VMEM((CHUNK, HIDDEN), jnp.float32),
            pltpu.VMEM((CHUNK, HIDDEN), jnp.float32),
            pltpu.VMEM((CHUNK, HIDDEN), jnp.float32),
            pltpu.VMEM((CHUNK, HIDDEN), jnp.float32),
            pltpu.VMEM((CHUNK, HIDDEN), jnp.float32),
            pltpu.SemaphoreType.DMA,
            pltpu.SemaphoreType.DMA,
            pltpu.SemaphoreType.DMA,
            pltpu.SemaphoreType.DMA,
            pltpu.SemaphoreType.DMA,
            pltpu.SemaphoreType.DMA,
        ],
    )
    return kern(token_flat, segment_flat, token_table, pos_table, seg_table)


def kernel(token, segment, token_table, pos_table, seg_table):
    token_flat = token.reshape(N).astype(jnp.int32)
    segment_flat = segment.reshape(N).astype(jnp.int32)
    seg_pad = jnp.concatenate(
        [seg_table, jnp.zeros((8 - SEG_NUM, HIDDEN), jnp.float32)], axis=0)
    out = _run(token_flat, segment_flat, token_table, pos_table, seg_pad)
    return out.reshape(B, L, HIDDEN)
